# baseline (speedup 1.0000x reference)
import functools

import jax
import jax.numpy as jnp
from jax import lax
from jax.experimental import pallas as pl
from jax.experimental.pallas import tpu as pltpu
from jax.experimental.pallas import tpu_sc as plsc

N_SRC = 10000
N_DST = 10000
E = 320000
D_FEAT = 128
D_EDGE = 16

NC = 2
NS = 16
NW = NC * NS
CH = 128
NCHUNK = E // CH
RCH = 128
NRCH = (N_DST + RCH - 1) // RCH
RTAIL = N_DST - (NRCH - 1) * RCH
NFULL = NCHUNK // NW * NW
JFULL = NCHUNK // NW


MAIN = 76


def _sc_body(y_hbm, m_hbm, sidx_hbm, didx_hbm, zg_hbm,
             t_out,
             s0, s1, s2, s3, d0, d1, d2, d3, r0, r1,
             t_sh, sl0, sl1, sl2, sl3, sg0, sg1, ss0, ss1):
    cid = lax.axis_index("c")
    sid = lax.axis_index("s")
    w = sid * NC + cid

    sidx = (s0, s1, s2, s3)
    didx = (d0, d1, d2, d3)
    rows = (r0, r1)
    sem_l = (sl0, sl1, sl2, sl3)
    sem_g = (sg0, sg1)
    sem_s = (ss0, ss1)

    pltpu.sync_copy(zg_hbm, r0)

    def zbody(j, carry):
        k = j * NS + sid
        z0 = k * RCH

        @pl.when(k < NRCH - 1)
        def _():
            pltpu.sync_copy(r0, t_sh.at[pl.ds(z0, RCH)])

        @pl.when(k == NRCH - 1)
        def _():
            pltpu.sync_copy(r0.at[pl.ds(0, RTAIL)],
                            t_sh.at[pl.ds(z0, RTAIL)])

        return carry

    lax.fori_loop(0, (NRCH + NS - 1) // NS, zbody, 0)
    plsc.subcore_barrier()

    mbufs = ((d0, r0, sl0, ss0), (d1, r1, sl1, ss1))

    def issue_loads_m(buf, c):
        dv, rv, sl, _ = buf
        e0 = c * CH
        pltpu.async_copy(didx_hbm.at[pl.ds(e0, CH)], dv, sl)
        pltpu.async_copy(m_hbm.at[pl.ds(e0, CH)], rv, sl)

    def wait_loads_m(buf, c):
        dv, rv, sl, _ = buf
        e0 = c * CH
        pltpu.make_async_copy(didx_hbm.at[pl.ds(e0, CH)], dv, sl).wait()
        pltpu.make_async_copy(m_hbm.at[pl.ds(e0, CH)], rv, sl).wait()

    issue_loads_m(mbufs[0], w)

    def mbody(u, carry):
        for b in (0, 1):
            buf = mbufs[b]
            j = 2 * u + b
            c = j * NW + w
            dv, rv, _, ss = buf

            @pl.when(j >= 2)
            def _():
                pltpu.make_async_copy(rv, t_sh.at[dv], ss).wait()

            wait_loads_m(buf, c)

            @pl.when(j + 1 < JFULL)
            def _():
                issue_loads_m(mbufs[1 - b], (j + 1) * NW + w)

            pltpu.async_copy(rv, t_sh.at[dv], ss, add=True)
        return carry

    lax.fori_loop(0, JFULL // 2, mbody, 0)
    pltpu.make_async_copy(r0, t_sh.at[d0], ss0).wait()
    pltpu.make_async_copy(r1, t_sh.at[d1], ss1).wait()

    @pl.when(w < NCHUNK - NFULL)
    def _():
        e0 = (NFULL + w) * CH
        pltpu.sync_copy(didx_hbm.at[pl.ds(e0, CH)], d0)
        pltpu.sync_copy(m_hbm.at[pl.ds(e0, CH)], r0)
        pltpu.sync_copy(r0, t_sh.at[d0], add=True)

    data_hbm = y_hbm

    def issue_loads(k, c):
        e0 = c * CH
        pltpu.async_copy(sidx_hbm.at[pl.ds(e0, CH)], sidx[k], sem_l[k])
        pltpu.async_copy(didx_hbm.at[pl.ds(e0, CH)], didx[k], sem_l[k])

    def wait_loads(k, c):
        e0 = c * CH
        pltpu.make_async_copy(
            sidx_hbm.at[pl.ds(e0, CH)], sidx[k], sem_l[k]).wait()
        pltpu.make_async_copy(
            didx_hbm.at[pl.ds(e0, CH)], didx[k], sem_l[k]).wait()

    for k in range(3):
        issue_loads(k, k * NW + w)
    wait_loads(0, w)
    pltpu.async_copy(data_hbm.at[sidx[0]], rows[0], sem_g[0])

    def ubody(u, carry):
        for b4 in range(4):
            j = 4 * u + b4
            c = j * NW + w
            br = b4 % 2
            kk = b4

            pltpu.make_async_copy(
                data_hbm.at[sidx[kk]], rows[br], sem_g[br]).wait()

            @pl.when(j >= 1)
            def _():
                pltpu.make_async_copy(
                    rows[1 - br], t_sh.at[didx[(kk + 3) % 4]],
                    sem_s[1 - br]).wait()

            @pl.when(j + 1 < MAIN)
            def _():
                wait_loads((kk + 1) % 4, c + NW)
                pltpu.async_copy(
                    data_hbm.at[sidx[(kk + 1) % 4]], rows[1 - br],
                    sem_g[1 - br])

            pltpu.async_copy(rows[br], t_sh.at[didx[kk]], sem_s[br],
                             add=True)

            @pl.when(j + 3 < MAIN)
            def _():
                issue_loads((kk + 3) % 4, c + 3 * NW)

        return carry

    lax.fori_loop(0, MAIN // 4, ubody, 0)
    pltpu.make_async_copy(rows[1], t_sh.at[didx[3]], sem_s[1]).wait()

    def tail(c):
        e0 = c * CH
        pltpu.sync_copy(sidx_hbm.at[pl.ds(e0, CH)], s0)
        pltpu.sync_copy(didx_hbm.at[pl.ds(e0, CH)], d0)
        pltpu.async_copy(data_hbm.at[s0], r0, sg0).wait()
        pltpu.sync_copy(r0, t_sh.at[d0], add=True)

    for j in range(MAIN, JFULL):
        tail(j * NW + w)

    @pl.when(w < NCHUNK - NFULL)
    def _():
        tail(NFULL + w)

    plsc.subcore_barrier()

    def obody(j, carry):
        k = j * NS + sid
        z0 = k * RCH
        ob = cid * N_DST + z0

        @pl.when(k < NRCH - 1)
        def _():
            pltpu.sync_copy(t_sh.at[pl.ds(z0, RCH)], r0)
            pltpu.sync_copy(r0, t_out.at[pl.ds(ob, RCH)])

        @pl.when(k == NRCH - 1)
        def _():
            pltpu.sync_copy(t_sh.at[pl.ds(z0, RTAIL)],
                            r0.at[pl.ds(0, RTAIL)])
            pltpu.sync_copy(r0.at[pl.ds(0, RTAIL)],
                            t_out.at[pl.ds(ob, RTAIL)])

        return carry

    lax.fori_loop(0, (NRCH + NS - 1) // NS, obody, 0)


_sc_call = functools.partial(
    pl.kernel,
    out_type=jax.ShapeDtypeStruct((NC * N_DST, D_FEAT), jnp.float32),
    mesh=plsc.VectorSubcoreMesh(core_axis_name="c", subcore_axis_name="s"),
    scratch_types=(
        [pltpu.VMEM((CH,), jnp.int32)] * 8
        + [pltpu.VMEM((CH, D_FEAT), jnp.float32)] * 2
        + [pltpu.VMEM_SHARED((N_DST, D_FEAT), jnp.float32)]
        + [pltpu.SemaphoreType.DMA] * 8
    ),
)(_sc_body)


def _y_body(xs, ws, y):
    y[...] = jnp.dot(xs[...], ws[...], preferred_element_type=jnp.float32)


def _m_body(eat, we, m):
    m[...] = lax.dot_general(
        eat[...], we[...], (((0,), (0,)), ((), ())),
        preferred_element_type=jnp.float32)


def _out_body(xd, t0, t1, out):
    out[...] = xd[...] + (t0[...] + t1[...])


_BM = 1000
_BE = 3200


def _tc_y(x_src, W_src):
    return pl.pallas_call(
        _y_body,
        grid=(N_SRC // _BM,),
        in_specs=[
            pl.BlockSpec((_BM, D_FEAT), lambda i: (i, 0)),
            pl.BlockSpec((D_FEAT, D_FEAT), lambda i: (0, 0)),
        ],
        out_specs=pl.BlockSpec((_BM, D_FEAT), lambda i: (i, 0)),
        out_shape=jax.ShapeDtypeStruct((N_SRC, D_FEAT), jnp.float32),
    )(x_src, W_src)


def _tc_m(edge_attr, W_edge):
    return pl.pallas_call(
        _m_body,
        grid=(E // _BE,),
        in_specs=[
            pl.BlockSpec((D_EDGE, _BE), lambda i: (0, i)),
            pl.BlockSpec((D_EDGE, D_FEAT), lambda i: (0, 0)),
        ],
        out_specs=pl.BlockSpec((_BE, D_FEAT), lambda i: (i, 0)),
        out_shape=jax.ShapeDtypeStruct((E, D_FEAT), jnp.float32),
    )(edge_attr.T, W_edge)


def _tc_post(x_dst, t0, t1):
    return pl.pallas_call(
        _out_body,
        grid=(N_DST // _BM,),
        in_specs=[pl.BlockSpec((_BM, D_FEAT), lambda i: (i, 0))] * 3,
        out_specs=pl.BlockSpec((_BM, D_FEAT), lambda i: (i, 0)),
        out_shape=jax.ShapeDtypeStruct((N_DST, D_FEAT), jnp.float32),
    )(x_dst, t0, t1)


def kernel(x_src, x_dst, edge_attr, edge_index, W_src, W_edge):
    src = edge_index[0].astype(jnp.int32)
    dst = edge_index[1].astype(jnp.int32)
    zg = jnp.zeros((CH, D_FEAT), jnp.float32)

    y = _tc_y(x_src, W_src)
    m = _tc_m(edge_attr, W_edge)
    t = _sc_call(y, m, src, dst, zg)

    return _tc_post(x_dst, t[:N_DST], t[N_DST:])


# speedup vs baseline: 1.0095x; 1.0095x over previous
import functools

import jax
import jax.numpy as jnp
from jax import lax
from jax.experimental import pallas as pl
from jax.experimental.pallas import tpu as pltpu
from jax.experimental.pallas import tpu_sc as plsc

N_SRC = 10000
N_DST = 10000
E = 320000
D_FEAT = 128
D_EDGE = 16

NC = 2
NS = 16
NW = NC * NS
CH = 128
NCHUNK = E // CH
RCH = 128
NRCH = (N_DST + RCH - 1) // RCH
RTAIL = N_DST - (NRCH - 1) * RCH
NFULL = NCHUNK // NW * NW
JFULL = NCHUNK // NW


def _sc_body(y_hbm, m_hbm, sidx_hbm, didx_hbm, zg_hbm,
             t_out,
             s0, s1, d0, d1, r0, r1,
             t_sh, sl0, sl1, sg0, sg1, ss0, ss1):
    cid = lax.axis_index("c")
    sid = lax.axis_index("s")
    w = sid * NC + cid

    rows = (r0, r1)
    sem_g = (sg0, sg1)
    sem_s = (ss0, ss1)

    pltpu.sync_copy(zg_hbm, r0)

    def zbody(j, carry):
        k = j * NS + sid
        z0 = k * RCH

        @pl.when(k < NRCH - 1)
        def _():
            pltpu.sync_copy(r0, t_sh.at[pl.ds(z0, RCH)])

        @pl.when(k == NRCH - 1)
        def _():
            pltpu.sync_copy(r0.at[pl.ds(0, RTAIL)],
                            t_sh.at[pl.ds(z0, RTAIL)])

        return carry

    lax.fori_loop(0, (NRCH + NS - 1) // NS, zbody, 0)
    plsc.subcore_barrier()

    mbufs = ((d0, r0, sl0, ss0), (d1, r1, sl1, ss1))

    def issue_loads_m(buf, c):
        dv, rv, sl, _ = buf
        e0 = c * CH
        pltpu.async_copy(didx_hbm.at[pl.ds(e0, CH)], dv, sl)
        pltpu.async_copy(m_hbm.at[pl.ds(e0, CH)], rv, sl)

    def wait_loads_m(buf, c):
        dv, rv, sl, _ = buf
        e0 = c * CH
        pltpu.make_async_copy(didx_hbm.at[pl.ds(e0, CH)], dv, sl).wait()
        pltpu.make_async_copy(m_hbm.at[pl.ds(e0, CH)], rv, sl).wait()

    issue_loads_m(mbufs[0], w)

    def mbody(u, carry):
        for b in (0, 1):
            buf = mbufs[b]
            j = 2 * u + b
            c = j * NW + w
            dv, rv, _, ss = buf

            @pl.when(j >= 2)
            def _():
                pltpu.make_async_copy(rv, t_sh.at[dv], ss).wait()

            wait_loads_m(buf, c)

            @pl.when(j + 1 < JFULL)
            def _():
                issue_loads_m(mbufs[1 - b], (j + 1) * NW + w)

            pltpu.async_copy(rv, t_sh.at[dv], ss, add=True)
        return carry

    lax.fori_loop(0, JFULL // 2, mbody, 0)
    pltpu.make_async_copy(r0, t_sh.at[d0], ss0).wait()
    pltpu.make_async_copy(r1, t_sh.at[d1], ss1).wait()

    @pl.when(w < NCHUNK - NFULL)
    def _():
        e0 = (NFULL + w) * CH
        pltpu.sync_copy(didx_hbm.at[pl.ds(e0, CH)], d0)
        pltpu.sync_copy(m_hbm.at[pl.ds(e0, CH)], r0)
        pltpu.sync_copy(r0, t_sh.at[d0], add=True)

    def issue_loads_y(buf, c):
        sv, dv, sl = buf
        e0 = c * CH
        pltpu.async_copy(sidx_hbm.at[pl.ds(e0, CH)], sv, sl)
        pltpu.async_copy(didx_hbm.at[pl.ds(e0, CH)], dv, sl)

    def wait_loads_y(buf, c):
        sv, dv, sl = buf
        e0 = c * CH
        pltpu.make_async_copy(sidx_hbm.at[pl.ds(e0, CH)], sv, sl).wait()
        pltpu.make_async_copy(didx_hbm.at[pl.ds(e0, CH)], dv, sl).wait()

    ybufs = ((s0, d0, sl0), (s1, d1, sl1))
    issue_loads_y(ybufs[0], w)

    def ybody(u, carry):
        for b in (0, 1):
            sv, dv, sl = ybufs[b]
            rv = rows[b]
            ss = sem_s[b]
            j = 2 * u + b
            c = j * NW + w

            @pl.when(j >= 2)
            def _():
                pltpu.make_async_copy(rv, t_sh.at[dv], ss).wait()

            wait_loads_y(ybufs[b], c)

            @pl.when(j + 1 < JFULL)
            def _():
                issue_loads_y(ybufs[1 - b], (j + 1) * NW + w)

            pltpu.async_copy(y_hbm.at[sv], rv, sem_g[b]).wait()
            pltpu.async_copy(rv, t_sh.at[dv], ss, add=True)
        return carry

    lax.fori_loop(0, JFULL // 2, ybody, 0)
    pltpu.make_async_copy(r0, t_sh.at[d0], ss0).wait()
    pltpu.make_async_copy(r1, t_sh.at[d1], ss1).wait()

    @pl.when(w < NCHUNK - NFULL)
    def _():
        c = NFULL + w
        e0 = c * CH
        pltpu.sync_copy(sidx_hbm.at[pl.ds(e0, CH)], s0)
        pltpu.sync_copy(didx_hbm.at[pl.ds(e0, CH)], d0)
        pltpu.async_copy(y_hbm.at[s0], r0, sg0).wait()
        pltpu.sync_copy(r0, t_sh.at[d0], add=True)

    plsc.subcore_barrier()

    def obody(j, carry):
        k = j * NS + sid
        z0 = k * RCH
        ob = cid * N_DST + z0

        @pl.when(k < NRCH - 1)
        def _():
            pltpu.sync_copy(t_sh.at[pl.ds(z0, RCH)], r0)
            pltpu.sync_copy(r0, t_out.at[pl.ds(ob, RCH)])

        @pl.when(k == NRCH - 1)
        def _():
            pltpu.sync_copy(t_sh.at[pl.ds(z0, RTAIL)],
                            r0.at[pl.ds(0, RTAIL)])
            pltpu.sync_copy(r0.at[pl.ds(0, RTAIL)],
                            t_out.at[pl.ds(ob, RTAIL)])

        return carry

    lax.fori_loop(0, (NRCH + NS - 1) // NS, obody, 0)


_sc_call = functools.partial(
    pl.kernel,
    out_type=jax.ShapeDtypeStruct((NC * N_DST, D_FEAT), jnp.float32),
    mesh=plsc.VectorSubcoreMesh(core_axis_name="c", subcore_axis_name="s"),
    scratch_types=(
        [pltpu.VMEM((CH,), jnp.int32)] * 4
        + [pltpu.VMEM((CH, D_FEAT), jnp.float32)] * 2
        + [pltpu.VMEM_SHARED((N_DST, D_FEAT), jnp.float32)]
        + [pltpu.SemaphoreType.DMA] * 6
    ),
)(_sc_body)


def _y_body(xs, ws, y):
    y[...] = jnp.dot(xs[...], ws[...], preferred_element_type=jnp.float32)


def _m_body(eat, we, m):
    m[...] = lax.dot_general(
        eat[...], we[...], (((0,), (0,)), ((), ())),
        preferred_element_type=jnp.float32)


def _out_body(xd, t0, t1, out):
    out[...] = xd[...] + (t0[...] + t1[...])


_BM = 1000
_BE = 3200


def _tc_y(x_src, W_src):
    return pl.pallas_call(
        _y_body,
        grid=(N_SRC // _BM,),
        in_specs=[
            pl.BlockSpec((_BM, D_FEAT), lambda i: (i, 0)),
            pl.BlockSpec((D_FEAT, D_FEAT), lambda i: (0, 0)),
        ],
        out_specs=pl.BlockSpec((_BM, D_FEAT), lambda i: (i, 0)),
        out_shape=jax.ShapeDtypeStruct((N_SRC, D_FEAT), jnp.float32),
    )(x_src, W_src)


def _tc_m(edge_attr, W_edge):
    return pl.pallas_call(
        _m_body,
        grid=(E // _BE,),
        in_specs=[
            pl.BlockSpec((D_EDGE, _BE), lambda i: (0, i)),
            pl.BlockSpec((D_EDGE, D_FEAT), lambda i: (0, 0)),
        ],
        out_specs=pl.BlockSpec((_BE, D_FEAT), lambda i: (i, 0)),
        out_shape=jax.ShapeDtypeStruct((E, D_FEAT), jnp.float32),
    )(edge_attr.T, W_edge)


def _tc_post(x_dst, t0, t1):
    return pl.pallas_call(
        _out_body,
        grid=(N_DST // _BM,),
        in_specs=[pl.BlockSpec((_BM, D_FEAT), lambda i: (i, 0))] * 3,
        out_specs=pl.BlockSpec((_BM, D_FEAT), lambda i: (i, 0)),
        out_shape=jax.ShapeDtypeStruct((N_DST, D_FEAT), jnp.float32),
    )(x_dst, t0, t1)


def kernel(x_src, x_dst, edge_attr, edge_index, W_src, W_edge):
    src = edge_index[0].astype(jnp.int32)
    dst = edge_index[1].astype(jnp.int32)
    zg = jnp.zeros((CH, D_FEAT), jnp.float32)

    y = _tc_y(x_src, W_src)
    m = _tc_m(edge_attr, W_edge)
    t = _sc_call(y, m, src, dst, zg)

    return _tc_post(x_dst, t[:N_DST], t[N_DST:])
